# Initial kernel scaffold; baseline (speedup 1.0000x reference)
import jax
import jax.numpy as jnp
from jax.experimental import pallas as pl


def kernel(x, en, edge_index, D_en, theta_W, theta_b, phi_W, phi_b, theta_en_params, phi_en_params, W_params):
    raise NotImplementedError("write your pallas kernel here")


# speedup vs baseline: 2.5285x; 2.5285x over previous
import functools

import jax
import jax.numpy as jnp
from jax import lax
from jax.experimental import pallas as pl
from jax.experimental.pallas import tpu as pltpu
from jax.experimental.pallas import tpu_sc as plsc

_BE = 2000
_RC = 512
_NBLK = 1024
_BIG = 1e30


def _mlp(h, params):
    for i, (W, b) in enumerate(params):
        h = jnp.dot(h, W, preferred_element_type=jnp.float32) + b
        if i < len(params) - 1:
            h = jnp.maximum(h, 0.0)
    return h


def _node_kernel(x_ref, en_ref, tW_ref, pW_ref, pb_ref, *pe_refs):
    src_ref = pe_refs[-1]
    pe = [(pe_refs[2 * i][...], pe_refs[2 * i + 1][...]) for i in range(4)]
    x = x_ref[...]
    en = en_ref[...]
    tx = jnp.dot(x, tW_ref[...], preferred_element_type=jnp.float32)
    px = jnp.dot(x, pW_ref[...], preferred_element_type=jnp.float32) + pb_ref[...]
    pen = _mlp(en, pe)
    z = jnp.zeros((x.shape[0], 88), jnp.float32)
    src_ref[...] = jnp.concatenate([tx, px, pen, en, z], axis=1)


def _sc_gather(table, idx):
    ep = idx.shape[0]
    d = table.shape[1]
    info = plsc.get_sparse_core_info()
    nw = info.num_cores * info.num_subcores
    b_per_w = ep // nw
    ch = 1000
    nch = b_per_w // ch
    mesh = plsc.VectorSubcoreMesh(core_axis_name="c", subcore_axis_name="s")

    @functools.partial(
        pl.kernel, mesh=mesh,
        out_type=jax.ShapeDtypeStruct((ep, d), jnp.float32),
        scratch_types=[
            pltpu.VMEM((ch,), jnp.int32),
            pltpu.VMEM((ch, d), jnp.float32),
            pltpu.SemaphoreType.DMA,
        ],
    )
    def k(table_hbm, idx_hbm, out_hbm, idx_v, rows_v, sem):
        wid = lax.axis_index("s") * info.num_cores + lax.axis_index("c")
        base = wid * b_per_w
        for c in range(nch):
            off = base + c * ch
            pltpu.sync_copy(idx_hbm.at[pl.ds(off, ch)], idx_v)
            pltpu.async_copy(table_hbm.at[idx_v], rows_v, sem).wait()
            pltpu.sync_copy(rows_v, out_hbm.at[pl.ds(off, ch)])

    return k(table, idx)


def _edge_kernel(nchunks, gs_ref, gd_ref, den_ref, dc_ref, dr_ref, tb_ref,
                 *w_refs):
    omax_ref, osum_ref = w_refs[-2], w_refs[-1]
    te = [(w_refs[2 * i][...], w_refs[2 * i + 1][...]) for i in range(4)]
    wp = [(w_refs[8 + 2 * i][...], w_refs[8 + 2 * i + 1][...]) for i in range(4)]

    i = pl.program_id(0)

    @pl.when(i == 0)
    def _init():
        omax_ref[...] = jnp.full(omax_ref.shape, -_BIG, jnp.float32)
        osum_ref[...] = jnp.zeros(osum_ref.shape, jnp.float32)

    gs = gs_ref[...]
    gd = gd_ref[...]
    tx_s, px_s = gs[:, 0:16], gs[:, 16:32]
    pen_s, en_s = gs[:, 32:36], gs[:, 36:40]
    tx_d, en_d = gd[:, 0:16], gd[:, 36:40]

    theta_x = tx_d - tx_s + tb_ref[...]
    phi_x = px_s
    theta_en = _mlp(en_d - en_s, te)
    phi_en = pen_s
    si = jnp.concatenate(
        [theta_x, phi_x, theta_en, phi_en, den_ref[...][:, 0:3]], axis=1)
    score = jax.nn.sigmoid(_mlp(si, wp))
    edge_x = theta_x + phi_x
    edge_en = phi_en + theta_en

    dc = dc_ref[...]
    be = dc.shape[0]
    val = edge_x
    s = 1
    while s < be:
        pv = jnp.concatenate(
            [jnp.full((s, 16), -_BIG, jnp.float32), val[:-s]], axis=0)
        pd = jnp.concatenate(
            [jnp.full((s, 1), -1, jnp.int32), dc[:-s]], axis=0)
        val = jnp.where(pd == dc, jnp.maximum(val, pv), val)
        s *= 2
    nxt = jnp.concatenate([dc[1:], jnp.full((1, 1), -1, jnp.int32)], axis=0)
    fin = (dc != nxt).astype(jnp.float32)

    ones = jnp.ones((be, 1), jnp.float32)
    zero = jnp.zeros((be, 1), jnp.float32)
    vals = jnp.concatenate(
        [val * fin, edge_en, score, ones, fin, zero], axis=1)

    dmin = dc[0, 0]
    dmax = dc[be - 1, 0]
    dr = dr_ref[...][0]

    for c in range(nchunks):
        lo = c * _RC

        @pl.when((dmax >= lo) & (dmin < lo + _RC))
        def _scatter(lo=lo):
            rowid = lax.broadcasted_iota(jnp.int32, (_RC, be), 0) + lo
            oh = (rowid == dr).astype(jnp.float32)
            contrib = jnp.dot(oh, vals, preferred_element_type=jnp.float32)
            osum_ref[lo:lo + _RC, :] = (
                osum_ref[lo:lo + _RC, :] + contrib[:, 16:24])
            ind = contrib[:, 22:23]
            cur = jnp.where(ind > 0.0, contrib[:, 0:16], -_BIG)
            omax_ref[lo:lo + _RC, :] = jnp.maximum(
                omax_ref[lo:lo + _RC, :], cur)


def _impl(x, en, edge_index, D_en, theta_W, theta_b, phi_W, phi_b,
          theta_en_params, phi_en_params, W_params):
    n = x.shape[0]
    e = edge_index.shape[1]
    np_ = ((n + _NBLK - 1) // _NBLK) * _NBLK
    ep = ((e + 32000 - 1) // 32000) * 32000
    nchunks = np_ // _RC

    x_p = jnp.pad(x, ((0, np_ - n), (0, 0)))
    en_p = jnp.pad(en, ((0, np_ - n), (0, 0)))
    pb2 = phi_b.reshape(1, -1)
    tb2 = theta_b.reshape(1, -1)
    pe_flat = [a for (W, b) in phi_en_params for a in (W, b.reshape(1, -1))]
    te_flat = [a for (W, b) in theta_en_params for a in (W, b.reshape(1, -1))]
    wp_flat = [a for (W, b) in W_params for a in (W, b.reshape(1, -1))]

    nb_n = np_ // _NBLK
    full = lambda shape: pl.BlockSpec(shape, lambda i: tuple(0 for _ in shape))
    node_t = pl.pallas_call(
        _node_kernel,
        grid=(nb_n,),
        in_specs=[
            pl.BlockSpec((_NBLK, x.shape[1]), lambda i: (i, 0)),
            pl.BlockSpec((_NBLK, en.shape[1]), lambda i: (i, 0)),
            full(theta_W.shape), full(phi_W.shape), full(pb2.shape),
        ] + [full(a.shape) for a in pe_flat],
        out_specs=pl.BlockSpec((_NBLK, 128), lambda i: (i, 0)),
        out_shape=jax.ShapeDtypeStruct((np_, 128), jnp.float32),
    )(x_p, en_p, theta_W, phi_W, pb2, *pe_flat)

    src = jnp.pad(edge_index[0], (0, ep - e))
    dst = jnp.pad(edge_index[1], (0, ep - e), constant_values=np_ - 1)
    den = jnp.pad(D_en, ((0, ep - e), (0, 5)))
    order = jnp.argsort(dst)
    src_s = src[order]
    dst_s = dst[order]
    den_s = den[order]

    g_src = _sc_gather(node_t, src_s)
    g_dst = _sc_gather(node_t, dst_s)

    nb = ep // _BE
    dc = dst_s.reshape(ep, 1)
    dr = dst_s.reshape(nb, 1, _BE)

    ws = [tb2] + te_flat + wp_flat
    out_max, out_sum = pl.pallas_call(
        functools.partial(_edge_kernel, nchunks),
        grid=(nb,),
        in_specs=[
            pl.BlockSpec((_BE, 128), lambda i: (i, 0)),
            pl.BlockSpec((_BE, 128), lambda i: (i, 0)),
            pl.BlockSpec((_BE, 8), lambda i: (i, 0)),
            pl.BlockSpec((_BE, 1), lambda i: (i, 0)),
            pl.BlockSpec((1, 1, _BE), lambda i: (i, 0, 0)),
        ] + [full(a.shape) for a in ws],
        out_specs=[
            pl.BlockSpec((np_, 16), lambda i: (0, 0)),
            pl.BlockSpec((np_, 8), lambda i: (0, 0)),
        ],
        out_shape=[
            jax.ShapeDtypeStruct((np_, 16), jnp.float32),
            jax.ShapeDtypeStruct((np_, 8), jnp.float32),
        ],
    )(g_src, g_dst, den_s, dc, dr, *ws)

    sums = out_sum[:n]
    cnt = sums[:, 5:6]
    denom = jnp.maximum(cnt, 1.0)
    x_out = jnp.where(cnt > 0, out_max[:n], 0.0)
    en_out = sums[:, 0:4] / denom
    score_n = sums[:, 4:5] / denom
    return x_out, en_out, score_n


_impl_jit = jax.jit(_impl)


def kernel(x, en, edge_index, D_en, theta_W, theta_b, phi_W, phi_b,
           theta_en_params, phi_en_params, W_params):
    return _impl_jit(x, en, edge_index, D_en, theta_W, theta_b, phi_W, phi_b,
                     theta_en_params, phi_en_params, W_params)
